# baseline (speedup 1.0000x reference)
import functools

import jax
import jax.numpy as jnp
from jax import lax
from jax.experimental import pallas as pl
from jax.experimental.pallas import tpu as pltpu
from jax.experimental.pallas import tpu_sc as plsc

NC, NS = 2, 16
NW = NC * NS
D = 64
CH = 128


@functools.lru_cache(maxsize=None)
def _make_gather(B):
    RW = B // NW
    NCH = RW // CH
    mesh = plsc.VectorSubcoreMesh(core_axis_name="c", subcore_axis_name="s")

    @functools.partial(
        pl.kernel,
        out_type=jax.ShapeDtypeStruct((B, D), jnp.float32),
        mesh=mesh,
        compiler_params=pltpu.CompilerParams(use_tc_tiling_on_sc=False),
        scratch_types=[
            pltpu.VMEM((NCH, CH), jnp.int32),
            pltpu.VMEM((CH, D), jnp.float32),
            pltpu.SemaphoreType.DMA,
        ],
    )
    def gather_kernel(table_hbm, idx_hbm, out_hbm, idx_v, rows_v, sem):
        wid = lax.axis_index("s") * NC + lax.axis_index("c")
        base = wid * RW
        pltpu.sync_copy(idx_hbm.at[wid], idx_v)

        @pl.loop(0, NCH)
        def _chunk(c):
            pltpu.async_copy(table_hbm.at[idx_v.at[c]], rows_v, sem).wait()
            pltpu.sync_copy(rows_v, out_hbm.at[pl.ds(base + c * CH, CH)])

    return gather_kernel


def kernel(indices, embed_weight):
    bdim, t = indices.shape
    b = bdim * t
    idx = indices.reshape(NW, b // NW // CH, CH)
    out = _make_gather(b)(embed_weight, idx)
    return out.reshape(bdim, t, D)


# speedup vs baseline: 1.1151x; 1.1151x over previous
import functools

import jax
import jax.numpy as jnp
from jax import lax
from jax.experimental import pallas as pl
from jax.experimental.pallas import tpu as pltpu
from jax.experimental.pallas import tpu_sc as plsc

NC, NS = 2, 16
NW = NC * NS
D = 64
CH = 128
NBUF = 8
LOOK = 4


@functools.lru_cache(maxsize=None)
def _make_gather(B):
    RW = B // NW
    NCH = RW // CH
    assert NCH % NBUF == 0 and NCH >= 2 * NBUF
    mesh = plsc.VectorSubcoreMesh(core_axis_name="c", subcore_axis_name="s")

    @functools.partial(
        pl.kernel,
        out_type=jax.ShapeDtypeStruct((B, D), jnp.float32),
        mesh=mesh,
        compiler_params=pltpu.CompilerParams(use_tc_tiling_on_sc=False),
        scratch_types=[
            pltpu.VMEM((NCH, CH), jnp.int32),
            pltpu.VMEM((NBUF, CH, D), jnp.float32),
            pltpu.SemaphoreType.DMA((NBUF,)),
            pltpu.SemaphoreType.DMA((NBUF,)),
        ],
    )
    def gather_kernel(table_hbm, idx_hbm, out_hbm, idx_v, rows_v, gsem, wsem):
        wid = lax.axis_index("s") * NC + lax.axis_index("c")
        base = wid * RW
        pltpu.sync_copy(idx_hbm.at[wid], idx_v)

        def fire_gather(c, j):
            pltpu.async_copy(table_hbm.at[idx_v.at[c]], rows_v.at[j],
                             gsem.at[j])

        def fire_writeback(c, j):
            pltpu.async_copy(rows_v.at[j],
                             out_hbm.at[pl.ds(base + c * CH, CH)], wsem.at[j])

        def wait_gather(j):
            pltpu.make_async_copy(out_hbm.at[pl.ds(0, CH)], rows_v.at[j],
                                  gsem.at[j]).wait()

        def wait_writeback(j):
            pltpu.make_async_copy(rows_v.at[j], out_hbm.at[pl.ds(0, CH)],
                                  wsem.at[j]).wait()

        for j in range(LOOK):
            fire_gather(j, j)

        for j in range(NBUF):
            wait_gather(j)
            fire_writeback(j, j)
            j2 = (j + LOOK) % NBUF
            if j >= LOOK:
                wait_writeback(j2)
            fire_gather(j + LOOK, j2)

        @pl.loop(NBUF, NCH - NBUF, step=NBUF)
        def _block(cb):
            for j in range(NBUF):
                c = cb + j
                wait_gather(j)
                fire_writeback(c, j)
                j2 = (j + LOOK) % NBUF
                wait_writeback(j2)
                fire_gather(c + LOOK, j2)

        for j in range(NBUF):
            c = NCH - NBUF + j
            wait_gather(j)
            fire_writeback(c, j)
            if j < NBUF - LOOK:
                j2 = (j + LOOK) % NBUF
                wait_writeback(j2)
                fire_gather(c + LOOK, j2)

        for j in range(NBUF):
            wait_writeback(j)

    return gather_kernel


def kernel(indices, embed_weight):
    bdim, t = indices.shape
    b = bdim * t
    idx = indices.reshape(NW, b // NW // CH, CH)
    out = _make_gather(b)(embed_weight, idx)
    return out.reshape(bdim, t, D)


# speedup vs baseline: 1.1166x; 1.0014x over previous
import functools

import jax
import jax.numpy as jnp
from jax import lax
from jax.experimental import pallas as pl
from jax.experimental.pallas import tpu as pltpu
from jax.experimental.pallas import tpu_sc as plsc

NC, NS = 2, 16
NW = NC * NS
NBUF = 4
LOOK = 2


@functools.lru_cache(maxsize=None)
def _make_gather(BT, T, D):
    RW = BT // NW
    assert RW % NBUF == 0 and RW >= 2 * NBUF
    T0 = min(128, T)
    T1 = T - T0
    mesh = plsc.VectorSubcoreMesh(core_axis_name="c", subcore_axis_name="s")

    @functools.partial(
        pl.kernel,
        out_type=jax.ShapeDtypeStruct((BT, T, D), jnp.float32),
        mesh=mesh,
        compiler_params=pltpu.CompilerParams(use_tc_tiling_on_sc=False),
        scratch_types=[
            pltpu.VMEM((RW, T), jnp.int32),
            pltpu.VMEM((NBUF, T, D), jnp.float32),
            pltpu.SemaphoreType.DMA((NBUF,)),
            pltpu.SemaphoreType.DMA((NBUF,)),
        ],
    )
    def gather_kernel(table_hbm, idx_hbm, out_hbm, idx_v, rows_v, gsem, wsem):
        wid = lax.axis_index("s") * NC + lax.axis_index("c")
        base = wid * RW
        pltpu.sync_copy(idx_hbm.at[pl.ds(base, RW)], idx_v)

        def fire_gather(r, j):
            pltpu.async_copy(table_hbm.at[idx_v.at[r, pl.ds(0, T0)]],
                             rows_v.at[j, pl.ds(0, T0)], gsem.at[j])
            if T1:
                pltpu.async_copy(table_hbm.at[idx_v.at[r, pl.ds(T0, T1)]],
                                 rows_v.at[j, pl.ds(T0, T1)], gsem.at[j])

        def fire_writeback(r, j):
            pltpu.async_copy(rows_v.at[j], out_hbm.at[base + r], wsem.at[j])

        def wait_gather(j):
            pltpu.make_async_copy(out_hbm.at[0], rows_v.at[j],
                                  gsem.at[j]).wait()

        def wait_writeback(j):
            pltpu.make_async_copy(rows_v.at[j], out_hbm.at[0],
                                  wsem.at[j]).wait()

        for j in range(LOOK):
            fire_gather(j, j)

        for j in range(NBUF):
            wait_gather(j)
            fire_writeback(j, j)
            j2 = (j + LOOK) % NBUF
            if j >= LOOK:
                wait_writeback(j2)
            fire_gather(j + LOOK, j2)

        @pl.loop(NBUF, RW - NBUF, step=NBUF)
        def _block(rb):
            for j in range(NBUF):
                r = rb + j
                wait_gather(j)
                fire_writeback(r, j)
                j2 = (j + LOOK) % NBUF
                wait_writeback(j2)
                fire_gather(r + LOOK, j2)

        for j in range(NBUF):
            r = RW - NBUF + j
            wait_gather(j)
            fire_writeback(r, j)
            if j < NBUF - LOOK:
                j2 = (j + LOOK) % NBUF
                wait_writeback(j2)
                fire_gather(r + LOOK, j2)

        for j in range(NBUF):
            wait_writeback(j)

    return gather_kernel


def kernel(indices, embed_weight):
    bt, t = indices.shape
    _, d = embed_weight.shape
    return _make_gather(bt, t, d)(embed_weight, indices)
